# baseline (speedup 1.0000x reference)
import functools

import jax
import jax.numpy as jnp
from jax import lax
from jax.experimental import pallas as pl
from jax.experimental.pallas import tpu as pltpu
from jax.experimental.pallas import tpu_sc as plsc

EMBED = 64
ROW_W = 128
CHUNKB = 128
NBUF = 2
BTILES = CHUNKB // 128


@functools.lru_cache(maxsize=None)
def _make_gather(batch: int, seq: int):
    info = plsc.get_sparse_core_info()
    nw = info.num_cores * info.num_subcores
    n_total = batch * seq
    per_w = n_total // nw
    n_chunks = per_w // CHUNKB
    q_per_l = batch // CHUNKB
    assert per_w * nw == n_total and n_chunks * CHUNKB == per_w
    assert n_chunks % NBUF == 0 and n_chunks // NBUF >= 3
    n_rounds = n_chunks // NBUF
    mesh = plsc.VectorSubcoreMesh(core_axis_name="c", subcore_axis_name="s")

    @functools.partial(
        pl.kernel,
        mesh=mesh,
        out_type=jax.ShapeDtypeStruct((seq, 8, batch // 128, 8, 128), jnp.float32),
        scratch_types=[
            pltpu.VMEM((per_w,), jnp.int32),
            pltpu.VMEM((NBUF, CHUNKB, ROW_W), jnp.float32),
            pltpu.VMEM((NBUF, 1, 8, BTILES, 8, 128), jnp.float32),
            pltpu.SemaphoreType.DMA((NBUF,)),
            pltpu.SemaphoreType.DMA((NBUF,)),
        ],
        compiler_params=pltpu.CompilerParams(
            use_tc_tiling_on_sc=False, needs_layout_passes=False),
    )
    def gather_kernel(idx_hbm, ptab_hbm, out_hbm, idx_all, grows, tiles,
                      gsem, ssem):
        wid = lax.axis_index("s") * info.num_cores + lax.axis_index("c")
        base = wid * per_w
        lane = lax.broadcasted_iota(jnp.int32, (16,), 0)

        def gather_start(clocal, nb):
            idx = idx_all.at[pl.ds(clocal * CHUNKB, CHUNKB)]
            pltpu.async_copy(ptab_hbm.at[idx], grows.at[nb], gsem.at[nb])

        def gather_wait(clocal, nb):
            idx = idx_all.at[pl.ds(clocal * CHUNKB, CHUNKB)]
            pltpu.make_async_copy(
                ptab_hbm.at[idx], grows.at[nb], gsem.at[nb]).wait()

        def out_slice(clocal):
            cglob = wid * n_chunks + clocal
            l = cglob // q_per_l
            bt0 = (cglob % q_per_l) * BTILES
            return out_hbm.at[pl.ds(l, 1), :, pl.ds(bt0, BTILES)]

        def store_start(clocal, nb):
            pltpu.async_copy(tiles.at[nb], out_slice(clocal), ssem.at[nb])

        def store_wait(clocal, nb):
            pltpu.make_async_copy(
                tiles.at[nb], out_slice(clocal), ssem.at[nb]).wait()

        def transpose(nb):
            g = grows.at[nb]
            for et in range(8):
                for bt in range(BTILES):
                    for el in range(8):
                        col = lane * 0 + (et * 8 + el)
                        for k in range(8):
                            rows = lane + (bt * 128 + k * 16)
                            v = plsc.load_gather(g, [rows, col])
                            tiles[nb, 0, et, bt, el, pl.ds(k * 16, 16)] = v

        pltpu.sync_copy(idx_hbm.at[pl.ds(base, per_w)], idx_all)
        for nb in range(NBUF):
            gather_start(nb, nb)

        def body(g, carry):
            c0 = g * NBUF
            for nb in range(NBUF):
                c = c0 + nb
                gather_wait(c, nb)

                @pl.when(g > 0)
                def _():
                    store_wait(c - NBUF, nb)

                transpose(nb)
                store_start(c, nb)

                @pl.when(g < n_rounds - 1)
                def _():
                    gather_start(c + NBUF, nb)

            return carry

        lax.fori_loop(0, n_rounds, body, 0)

        cf = (n_rounds - 1) * NBUF
        for nb in range(NBUF):
            store_wait(cf + nb, nb)

    return gather_kernel


def kernel(x, table):
    b, l = x.shape
    flat = x.T.reshape(b * l).astype(jnp.int32)
    padded = jnp.pad(table, ((0, 0), (0, ROW_W - EMBED)))
    out5d = _make_gather(b, l)(flat, padded)
    return out5d.transpose(2, 4, 0, 1, 3).reshape(b, l, EMBED)


# speedup vs baseline: 1.6350x; 1.6350x over previous
import functools

import jax
import jax.numpy as jnp
from jax import lax
from jax.experimental import pallas as pl
from jax.experimental.pallas import tpu as pltpu
from jax.experimental.pallas import tpu_sc as plsc

EMBED = 64
CHUNK = 512
NBUF = 2


@functools.lru_cache(maxsize=None)
def _make_gather(n_total: int):
    info = plsc.get_sparse_core_info()
    nw = info.num_cores * info.num_subcores
    per_w = n_total // nw
    assert per_w * nw == n_total and per_w % (CHUNK * NBUF) == 0
    n_rounds = per_w // (CHUNK * NBUF)
    assert n_rounds >= 2
    mesh = plsc.VectorSubcoreMesh(core_axis_name="c", subcore_axis_name="s")

    @functools.partial(
        pl.kernel,
        mesh=mesh,
        out_type=jax.ShapeDtypeStruct((n_total, EMBED), jnp.float32),
        scratch_types=[
            pltpu.VMEM((per_w,), jnp.int32),
            pltpu.VMEM((NBUF, CHUNK, EMBED), jnp.float32),
            pltpu.SemaphoreType.DMA((NBUF,)),
            pltpu.SemaphoreType.DMA((NBUF,)),
        ],
        compiler_params=pltpu.CompilerParams(use_tc_tiling_on_sc=False),
    )
    def gather_kernel(idx_hbm, table_hbm, out_hbm, idx_all, rows, gsem, ssem):
        wid = lax.axis_index("s") * info.num_cores + lax.axis_index("c")
        base = wid * per_w

        def idx_slice(c):
            return idx_all.at[pl.ds(c * CHUNK, CHUNK)]

        def out_slice(c):
            return out_hbm.at[pl.ds(base + c * CHUNK, CHUNK)]

        pltpu.sync_copy(idx_hbm.at[pl.ds(base, per_w)], idx_all)
        for b in range(NBUF):
            pltpu.async_copy(table_hbm.at[idx_slice(b)], rows.at[b], gsem.at[b])

        def body(g, carry):
            c0 = g * NBUF
            for b in range(NBUF):
                pltpu.make_async_copy(
                    table_hbm.at[idx_slice(c0 + b)], rows.at[b], gsem.at[b]
                ).wait()
                pltpu.async_copy(rows.at[b], out_slice(c0 + b), ssem.at[b])
            for b in range(NBUF):
                pltpu.make_async_copy(
                    rows.at[b], out_slice(c0 + b), ssem.at[b]
                ).wait()
                pltpu.async_copy(
                    table_hbm.at[idx_slice(c0 + NBUF + b)], rows.at[b], gsem.at[b]
                )
            return carry

        lax.fori_loop(0, n_rounds - 1, body, 0)

        cf = (n_rounds - 1) * NBUF
        for b in range(NBUF):
            pltpu.make_async_copy(
                table_hbm.at[idx_slice(cf + b)], rows.at[b], gsem.at[b]
            ).wait()
            pltpu.async_copy(rows.at[b], out_slice(cf + b), ssem.at[b])
        for b in range(NBUF):
            pltpu.make_async_copy(rows.at[b], out_slice(cf + b), ssem.at[b]).wait()

    return gather_kernel


def kernel(x, table):
    b, l = x.shape
    flat = x.T.reshape(b * l).astype(jnp.int32)
    out = _make_gather(b * l)(flat, table)
    return out.reshape(l, b, EMBED).transpose(1, 0, 2)
